# Initial kernel scaffold; baseline (speedup 1.0000x reference)
import jax
import jax.numpy as jnp
from jax.experimental import pallas as pl


def kernel(pos, batch, pw0, pb0, pw1, pb1, aw0, ab0, lin_w, lin_src_w, lin_dst_w):
    raise NotImplementedError("write your pallas kernel here")


# speedup vs baseline: 12.1401x; 12.1401x over previous
import functools

import jax
import jax.numpy as jnp
from jax import lax
from jax.experimental import pallas as pl
from jax.experimental.pallas import tpu as pltpu

NPTS = 10000
NPAD = 10240
NSAMP = 5000
NSPAD = 5120
RAD2 = 0.25 * 0.25
KNB = 10
HID = 64
OCH = 128
QBLK = 256
CBLK = 512


def _fps_body(p3_ref, idx_ref, d_ref):
    px = p3_ref[0]
    py = p3_ref[1]
    pz = p3_ref[2]
    rows = lax.broadcasted_iota(jnp.int32, (8, 1280), 0)
    cols = lax.broadcasted_iota(jnp.int32, (8, 1280), 1)
    fidx = rows * 1280 + cols
    inb = fidx < NPTS

    sel0 = fidx == 0
    x0 = jnp.sum(jnp.where(sel0, px, 0.0))
    y0 = jnp.sum(jnp.where(sel0, py, 0.0))
    z0 = jnp.sum(jnp.where(sel0, pz, 0.0))
    dx = px - x0
    dy = py - y0
    dz = pz - z0
    d0 = (dx * dx + dy * dy) + dz * dz
    d_ref[...] = jnp.where(inb, d0, -jnp.inf)
    idx_ref[0] = jnp.int32(0)

    def body(i, carry):
        d = d_ref[...]
        m = jnp.max(d)
        eq = d == m
        nxt = jnp.min(jnp.where(eq, fidx, jnp.int32(NPAD)))
        sel = fidx == nxt
        xs = jnp.sum(jnp.where(sel, px, 0.0))
        ys = jnp.sum(jnp.where(sel, py, 0.0))
        zs = jnp.sum(jnp.where(sel, pz, 0.0))
        ddx = px - xs
        ddy = py - ys
        ddz = pz - zs
        nd = (ddx * ddx + ddy * ddy) + ddz * ddz
        d_ref[...] = jnp.minimum(d, nd)
        idx_ref[i] = nxt
        return carry

    lax.fori_loop(1, NSAMP, body, 0)


def _fps(pos):
    p = jnp.zeros((3, NPAD), jnp.float32).at[:, :NPTS].set(pos.T)
    p3 = p.reshape(3, 8, 1280)
    idx = pl.pallas_call(
        _fps_body,
        out_shape=jax.ShapeDtypeStruct((NSAMP,), jnp.int32),
        in_specs=[pl.BlockSpec(memory_space=pltpu.VMEM)],
        out_specs=pl.BlockSpec(memory_space=pltpu.SMEM),
        scratch_shapes=[pltpu.VMEM((8, 1280), jnp.float32)],
    )(p3)
    return idx


def _knn_body(qp_ref, pt_ref, cols_ref, ev_ref, d2_ref):
    qp = qp_ref[...]
    pt = pt_ref[...]
    q2 = jnp.sum(qp * qp, axis=1, keepdims=True)
    p2 = jnp.sum(pt * pt, axis=0, keepdims=True)
    dot = jnp.dot(qp, pt, preferred_element_type=jnp.float32)
    d2 = q2 + p2 - 2.0 * dot
    cidx = lax.broadcasted_iota(jnp.int32, (QBLK, NPAD), 1)
    valid = (d2 <= RAD2) & (cidx < NPTS)
    d2_ref[...] = jnp.where(valid, d2, jnp.inf)

    for k in range(KNB):
        d2m = d2_ref[...]
        mn = jnp.min(d2m, axis=1, keepdims=True)
        eq = d2m == mn
        c = jnp.min(jnp.where(eq, cidx, jnp.int32(NPAD)), axis=1,
                    keepdims=True)
        cols_ref[:, k:k + 1] = c
        ev_ref[:, k:k + 1] = (mn < jnp.inf).astype(jnp.int32)
        d2_ref[...] = jnp.where(cidx == c, jnp.inf, d2m)


def _knn(qpos16, post16):
    grid = NSPAD // QBLK
    cols, ev = pl.pallas_call(
        _knn_body,
        grid=(grid,),
        in_specs=[
            pl.BlockSpec((QBLK, 16), lambda b: (b, 0)),
            pl.BlockSpec((16, NPAD), lambda b: (0, 0)),
        ],
        out_specs=[
            pl.BlockSpec((QBLK, 16), lambda b: (b, 0)),
            pl.BlockSpec((QBLK, 16), lambda b: (b, 0)),
        ],
        out_shape=[
            jax.ShapeDtypeStruct((NSPAD, 16), jnp.int32),
            jax.ShapeDtypeStruct((NSPAD, 16), jnp.int32),
        ],
        scratch_shapes=[pltpu.VMEM((QBLK, NPAD), jnp.float32)],
    )(qpos16, post16)
    return cols, ev


def _conv_body(qp_ref, pc_ref, ev_ref,
               pw0_ref, pb0_ref, pw1_ref, pb1_ref, aw0_ref, ab0_ref,
               lw_ref, lsw_ref, ldw_ref, out_ref):
    qp = qp_ref[...]
    pw0 = pw0_ref[...]
    pb0 = pb0_ref[...]
    pw1 = pw1_ref[...]
    pb1 = pb1_ref[...]
    aw0 = aw0_ref[...]
    ab0 = ab0_ref[...]
    lw = lw_ref[...]
    lsw = lsw_ref[...]
    ldw = ldw_ref[...]

    adst = jnp.dot(qp, ldw, preferred_element_type=jnp.float32)

    alphas = []
    deltas = []
    xvs = []
    amax = None
    for j in range(KNB):
        pc = pc_ref[j]
        asrc = jnp.dot(pc, lsw, preferred_element_type=jnp.float32)
        rel = qp - pc
        h = jnp.maximum(
            jnp.dot(rel, pw0, preferred_element_type=jnp.float32) + pb0, 0.0)
        delta = jnp.dot(h, pw1, preferred_element_type=jnp.float32) + pb1
        a = jnp.dot(adst - asrc + delta, aw0,
                    preferred_element_type=jnp.float32) + ab0
        evj = ev_ref[:, j:j + 1] > 0
        a = jnp.where(evj, a, -1e30)
        xv = jnp.dot(pc, lw, preferred_element_type=jnp.float32)
        alphas.append(a)
        deltas.append(delta)
        xvs.append(xv)
        amax = a if amax is None else jnp.maximum(amax, a)

    amax = jnp.where(amax > -1e29, amax, 0.0)
    den = jnp.zeros_like(amax)
    acc = jnp.zeros_like(amax)
    for j in range(KNB):
        ex = jnp.exp(alphas[j] - amax)
        den = den + ex
        acc = acc + ex * (xvs[j] + deltas[j])
    out_ref[...] = acc / jnp.maximum(den, 1e-16)


def _conv(qpos16, posc, ev, pw0p, pb0, pw1, pb1, aw0, ab0, lwp, lswp, ldwp):
    grid = NSPAD // CBLK
    out = pl.pallas_call(
        _conv_body,
        grid=(grid,),
        in_specs=[
            pl.BlockSpec((CBLK, 16), lambda b: (b, 0)),
            pl.BlockSpec((KNB, CBLK, 16), lambda b: (0, b, 0)),
            pl.BlockSpec((CBLK, 16), lambda b: (b, 0)),
            pl.BlockSpec((16, HID), lambda b: (0, 0)),
            pl.BlockSpec((1, HID), lambda b: (0, 0)),
            pl.BlockSpec((HID, OCH), lambda b: (0, 0)),
            pl.BlockSpec((1, OCH), lambda b: (0, 0)),
            pl.BlockSpec((OCH, OCH), lambda b: (0, 0)),
            pl.BlockSpec((1, OCH), lambda b: (0, 0)),
            pl.BlockSpec((16, OCH), lambda b: (0, 0)),
            pl.BlockSpec((16, OCH), lambda b: (0, 0)),
            pl.BlockSpec((16, OCH), lambda b: (0, 0)),
        ],
        out_specs=pl.BlockSpec((CBLK, OCH), lambda b: (b, 0)),
        out_shape=jax.ShapeDtypeStruct((NSPAD, OCH), jnp.float32),
    )(qpos16, posc, ev, pw0p, pb0, pw1, pb1, aw0, ab0, lwp, lswp, ldwp)
    return out


def kernel(pos, batch, pw0, pb0, pw1, pb1, aw0, ab0, lin_w, lin_src_w,
           lin_dst_w):
    pos = pos.astype(jnp.float32)

    idx = _fps(pos)

    pos16 = jnp.zeros((NPAD, 16), jnp.float32).at[:NPTS, :3].set(pos)
    post16 = pos16.T
    idx_pad = jnp.zeros((NSPAD,), jnp.int32).at[:NSAMP].set(idx)
    qpos16 = pos16[idx_pad]
    cols16, ev16 = _knn(qpos16, post16)

    cols_jq = cols16[:, :KNB].T.reshape(-1)
    posc_flat = pos16[cols_jq]
    posc = posc_flat.reshape(KNB, NSPAD, 16)

    qp_conv = jnp.zeros((NSPAD, 16), jnp.float32).at[:, :3].set(pos[:NSPAD])
    pw0p = jnp.zeros((16, HID), jnp.float32).at[:3].set(pw0)
    lwp = jnp.zeros((16, OCH), jnp.float32).at[:3].set(lin_w)
    lswp = jnp.zeros((16, OCH), jnp.float32).at[:3].set(lin_src_w)
    ldwp = jnp.zeros((16, OCH), jnp.float32).at[:3].set(lin_dst_w)
    res = _conv(qp_conv, posc, ev16, pw0p, pb0[None, :], pw1, pb1[None, :],
                aw0, ab0[None, :], lwp, lswp, ldwp)

    out = jnp.zeros((NPTS, OCH), jnp.float32).at[:NSAMP].set(res[:NSAMP])
    return out


# speedup vs baseline: 12.7051x; 1.0465x over previous
import functools

import jax
import jax.numpy as jnp
from jax import lax
from jax.experimental import pallas as pl
from jax.experimental.pallas import tpu as pltpu
from jax.experimental.pallas import tpu_sc as plsc

NPTS = 10000
NPAD = 10240
NSAMP = 5000
NSPAD = 5120
RAD2 = 0.25 * 0.25
KNB = 10
HID = 64
OCH = 128
QBLK = 256
CBLK = 512


def _fps_body(p3_ref, idx_ref, d_ref):
    px = p3_ref[0]
    py = p3_ref[1]
    pz = p3_ref[2]
    rows = lax.broadcasted_iota(jnp.int32, (8, 1280), 0)
    cols = lax.broadcasted_iota(jnp.int32, (8, 1280), 1)
    fidx = rows * 1280 + cols
    inb = fidx < NPTS

    sel0 = fidx == 0
    x0 = jnp.sum(jnp.where(sel0, px, 0.0))
    y0 = jnp.sum(jnp.where(sel0, py, 0.0))
    z0 = jnp.sum(jnp.where(sel0, pz, 0.0))
    dx = px - x0
    dy = py - y0
    dz = pz - z0
    d0 = (dx * dx + dy * dy) + dz * dz
    d_ref[...] = jnp.where(inb, d0, -jnp.inf)
    idx_ref[0] = jnp.int32(0)

    def body(i, carry):
        d = d_ref[...]
        m = jnp.max(d)
        eq = d == m
        nxt = jnp.min(jnp.where(eq, fidx, jnp.int32(NPAD)))
        sel = fidx == nxt
        xs = jnp.sum(jnp.where(sel, px, 0.0))
        ys = jnp.sum(jnp.where(sel, py, 0.0))
        zs = jnp.sum(jnp.where(sel, pz, 0.0))
        ddx = px - xs
        ddy = py - ys
        ddz = pz - zs
        nd = (ddx * ddx + ddy * ddy) + ddz * ddz
        d_ref[...] = jnp.minimum(d, nd)
        idx_ref[i] = nxt
        return carry

    lax.fori_loop(1, NSAMP, body, 0)


def _fps(pos):
    p = jnp.zeros((3, NPAD), jnp.float32).at[:, :NPTS].set(pos.T)
    p3 = p.reshape(3, 8, 1280)
    idx = pl.pallas_call(
        _fps_body,
        out_shape=jax.ShapeDtypeStruct((NSAMP,), jnp.int32),
        in_specs=[pl.BlockSpec(memory_space=pltpu.VMEM)],
        out_specs=pl.BlockSpec(memory_space=pltpu.SMEM),
        scratch_shapes=[pltpu.VMEM((8, 1280), jnp.float32)],
    )(p3)
    return idx


def _knn_body(qp_ref, pt_ref, cols_ref, ev_ref, d2_ref):
    qp = qp_ref[...][:, :16]
    pt = pt_ref[...]
    q2 = jnp.sum(qp * qp, axis=1, keepdims=True)
    p2 = jnp.sum(pt * pt, axis=0, keepdims=True)
    dot = jnp.dot(qp, pt, preferred_element_type=jnp.float32)
    d2 = q2 + p2 - 2.0 * dot
    cidx = lax.broadcasted_iota(jnp.int32, (QBLK, NPAD), 1)
    valid = (d2 <= RAD2) & (cidx < NPTS)
    d2_ref[...] = jnp.where(valid, d2, jnp.inf)

    for k in range(KNB):
        d2m = d2_ref[...]
        mn = jnp.min(d2m, axis=1, keepdims=True)
        eq = d2m == mn
        c = jnp.min(jnp.where(eq, cidx, jnp.int32(NPAD)), axis=1,
                    keepdims=True)
        cols_ref[:, k:k + 1] = c
        ev_ref[:, k:k + 1] = (mn < jnp.inf).astype(jnp.int32)
        d2_ref[...] = jnp.where(cidx == c, jnp.inf, d2m)


def _knn(qpos16, post16):
    grid = NSPAD // QBLK
    cols, ev = pl.pallas_call(
        _knn_body,
        grid=(grid,),
        in_specs=[
            pl.BlockSpec((QBLK, 128), lambda b: (b, 0)),
            pl.BlockSpec((16, NPAD), lambda b: (0, 0)),
        ],
        out_specs=[
            pl.BlockSpec((QBLK, 16), lambda b: (b, 0)),
            pl.BlockSpec((QBLK, 16), lambda b: (b, 0)),
        ],
        out_shape=[
            jax.ShapeDtypeStruct((NSPAD, 16), jnp.int32),
            jax.ShapeDtypeStruct((NSPAD, 16), jnp.int32),
        ],
        scratch_shapes=[pltpu.VMEM((QBLK, NPAD), jnp.float32)],
    )(qpos16, post16)
    return cols, ev


def _sc_gather(table, idx):
    B = idx.shape[0]
    bpw = B // 32
    nchunks = max(1, bpw // 400)
    csz = bpw // nchunks
    mesh = plsc.VectorSubcoreMesh(core_axis_name="c", subcore_axis_name="s")

    @functools.partial(
        pl.kernel, mesh=mesh,
        out_type=jax.ShapeDtypeStruct((B, 128), jnp.float32),
        scratch_types=[
            pltpu.VMEM((bpw,), jnp.int32),
            pltpu.VMEM((csz, 128), jnp.float32),
            pltpu.SemaphoreType.DMA,
        ],
    )
    def k(table_hbm, idx_hbm, out_hbm, idx_v, rows_v, sem):
        wid = lax.axis_index("s") * 2 + lax.axis_index("c")
        base = wid * bpw
        pltpu.sync_copy(idx_hbm.at[pl.ds(base, bpw)], idx_v)
        for c in range(nchunks):
            pltpu.async_copy(table_hbm.at[idx_v.at[pl.ds(c * csz, csz)]],
                             rows_v, sem).wait()
            pltpu.sync_copy(rows_v, out_hbm.at[pl.ds(base + c * csz, csz)])

    return k(table, idx)


def _conv_body(qp_ref, pc_ref, ev_ref,
               pw0_ref, pb0_ref, pw1_ref, pb1_ref, aw0_ref, ab0_ref,
               lw_ref, lsw_ref, ldw_ref, out_ref):
    qp = qp_ref[...]
    pw0 = pw0_ref[...]
    pb0 = pb0_ref[...]
    pw1 = pw1_ref[...]
    pb1 = pb1_ref[...]
    aw0 = aw0_ref[...]
    ab0 = ab0_ref[...]
    lw = lw_ref[...]
    lsw = lsw_ref[...]
    ldw = ldw_ref[...]

    adst = jnp.dot(qp, ldw, preferred_element_type=jnp.float32)

    alphas = []
    deltas = []
    xvs = []
    amax = None
    for j in range(KNB):
        pc = pc_ref[j][:, :16]
        asrc = jnp.dot(pc, lsw, preferred_element_type=jnp.float32)
        rel = qp - pc
        h = jnp.maximum(
            jnp.dot(rel, pw0, preferred_element_type=jnp.float32) + pb0, 0.0)
        delta = jnp.dot(h, pw1, preferred_element_type=jnp.float32) + pb1
        a = jnp.dot(adst - asrc + delta, aw0,
                    preferred_element_type=jnp.float32) + ab0
        evj = ev_ref[:, j:j + 1] > 0
        a = jnp.where(evj, a, -1e30)
        xv = jnp.dot(pc, lw, preferred_element_type=jnp.float32)
        alphas.append(a)
        deltas.append(delta)
        xvs.append(xv)
        amax = a if amax is None else jnp.maximum(amax, a)

    amax = jnp.where(amax > -1e29, amax, 0.0)
    den = jnp.zeros_like(amax)
    acc = jnp.zeros_like(amax)
    for j in range(KNB):
        ex = jnp.exp(alphas[j] - amax)
        den = den + ex
        acc = acc + ex * (xvs[j] + deltas[j])
    out_ref[...] = acc / jnp.maximum(den, 1e-16)


def _conv(qpos16, posc, ev, pw0p, pb0, pw1, pb1, aw0, ab0, lwp, lswp, ldwp):
    grid = NSPAD // CBLK
    out = pl.pallas_call(
        _conv_body,
        grid=(grid,),
        in_specs=[
            pl.BlockSpec((CBLK, 16), lambda b: (b, 0)),
            pl.BlockSpec((KNB, CBLK, 128), lambda b: (0, b, 0)),
            pl.BlockSpec((CBLK, 16), lambda b: (b, 0)),
            pl.BlockSpec((16, HID), lambda b: (0, 0)),
            pl.BlockSpec((1, HID), lambda b: (0, 0)),
            pl.BlockSpec((HID, OCH), lambda b: (0, 0)),
            pl.BlockSpec((1, OCH), lambda b: (0, 0)),
            pl.BlockSpec((OCH, OCH), lambda b: (0, 0)),
            pl.BlockSpec((1, OCH), lambda b: (0, 0)),
            pl.BlockSpec((16, OCH), lambda b: (0, 0)),
            pl.BlockSpec((16, OCH), lambda b: (0, 0)),
            pl.BlockSpec((16, OCH), lambda b: (0, 0)),
        ],
        out_specs=pl.BlockSpec((CBLK, OCH), lambda b: (b, 0)),
        out_shape=jax.ShapeDtypeStruct((NSPAD, OCH), jnp.float32),
    )(qpos16, posc, ev, pw0p, pb0, pw1, pb1, aw0, ab0, lwp, lswp, ldwp)
    return out


def kernel(pos, batch, pw0, pb0, pw1, pb1, aw0, ab0, lin_w, lin_src_w,
           lin_dst_w):
    pos = pos.astype(jnp.float32)

    idx = _fps(pos)

    pos128 = jnp.zeros((NPAD, 128), jnp.float32).at[:NPTS, :3].set(pos)
    post16 = jnp.zeros((16, NPAD), jnp.float32).at[:3, :NPTS].set(pos.T)
    idx_pad = jnp.zeros((NSPAD,), jnp.int32).at[:NSAMP].set(idx)
    qpos16 = _sc_gather(pos128, idx_pad)
    cols16, ev16 = _knn(qpos16, post16)

    cols_jq = cols16[:, :KNB].T.reshape(-1)
    posc = _sc_gather(pos128, cols_jq).reshape(KNB, NSPAD, 128)

    qp_conv = jnp.zeros((NSPAD, 16), jnp.float32).at[:, :3].set(pos[:NSPAD])
    pw0p = jnp.zeros((16, HID), jnp.float32).at[:3].set(pw0)
    lwp = jnp.zeros((16, OCH), jnp.float32).at[:3].set(lin_w)
    lswp = jnp.zeros((16, OCH), jnp.float32).at[:3].set(lin_src_w)
    ldwp = jnp.zeros((16, OCH), jnp.float32).at[:3].set(lin_dst_w)
    res = _conv(qp_conv, posc, ev16, pw0p, pb0[None, :], pw1, pb1[None, :],
                aw0, ab0[None, :], lwp, lswp, ldwp)

    out = jnp.zeros((NPTS, OCH), jnp.float32).at[:NSAMP].set(res[:NSAMP])
    return out


# speedup vs baseline: 15.8893x; 1.2506x over previous
import functools

import jax
import jax.numpy as jnp
from jax import lax
from jax.experimental import pallas as pl
from jax.experimental.pallas import tpu as pltpu
from jax.experimental.pallas import tpu_sc as plsc

NPTS = 10000
NPAD = 10240
NSAMP = 5000
NSPAD = 5120
RAD2 = 0.25 * 0.25
KNB = 10
HID = 64
OCH = 128
QBLK = 256
CBLK = 512


def _fps_body(p3_ref, ps_ref, idx_ref, d_ref):
    px = p3_ref[0]
    py = p3_ref[1]
    pz = p3_ref[2]
    rows = lax.broadcasted_iota(jnp.int32, (8, 1280), 0)
    cols = lax.broadcasted_iota(jnp.int32, (8, 1280), 1)
    fidx = rows * 1280 + cols
    inb = fidx < NPTS

    x0 = ps_ref[0, 0]
    y0 = ps_ref[1, 0]
    z0 = ps_ref[2, 0]
    dx = px - x0
    dy = py - y0
    dz = pz - z0
    d0 = (dx * dx + dy * dy) + dz * dz
    d_ref[...] = jnp.where(inb, d0, -jnp.inf)
    idx_ref[0] = jnp.int32(0)

    def body(i, carry):
        d = d_ref[...]
        m = jnp.max(d)
        eq = d == m
        nxt = jnp.min(jnp.where(eq, fidx, jnp.int32(NPAD)))
        xs = ps_ref[0, nxt]
        ys = ps_ref[1, nxt]
        zs = ps_ref[2, nxt]
        ddx = px - xs
        ddy = py - ys
        ddz = pz - zs
        nd = (ddx * ddx + ddy * ddy) + ddz * ddz
        d_ref[...] = jnp.minimum(d, nd)
        idx_ref[i] = nxt
        return carry

    lax.fori_loop(1, NSAMP, body, 0)


def _fps(pos):
    p = jnp.zeros((3, NPAD), jnp.float32).at[:, :NPTS].set(pos.T)
    p3 = p.reshape(3, 8, 1280)
    idx = pl.pallas_call(
        _fps_body,
        out_shape=jax.ShapeDtypeStruct((NSAMP,), jnp.int32),
        in_specs=[pl.BlockSpec(memory_space=pltpu.VMEM),
                  pl.BlockSpec(memory_space=pltpu.SMEM)],
        out_specs=pl.BlockSpec(memory_space=pltpu.SMEM),
        scratch_shapes=[pltpu.VMEM((8, 1280), jnp.float32)],
    )(p3, p)
    return idx


def _knn_body(qp_ref, pt_ref, cols_ref, ev_ref, d2_ref):
    qp = qp_ref[...][:, :16]
    pt = pt_ref[...]
    q2 = jnp.sum(qp * qp, axis=1, keepdims=True)
    p2 = jnp.sum(pt * pt, axis=0, keepdims=True)
    dot = jnp.dot(qp, pt, preferred_element_type=jnp.float32)
    d2 = q2 + p2 - 2.0 * dot
    cidx = lax.broadcasted_iota(jnp.int32, (QBLK, NPAD), 1)
    valid = (d2 <= RAD2) & (cidx < NPTS)
    d2_ref[...] = jnp.where(valid, d2, jnp.inf)

    for k in range(KNB):
        d2m = d2_ref[...]
        mn = jnp.min(d2m, axis=1, keepdims=True)
        eq = d2m == mn
        c = jnp.min(jnp.where(eq, cidx, jnp.int32(NPAD)), axis=1,
                    keepdims=True)
        cols_ref[:, k:k + 1] = c
        ev_ref[:, k:k + 1] = (mn < jnp.inf).astype(jnp.int32)
        d2_ref[...] = jnp.where(cidx == c, jnp.inf, d2m)


def _knn(qpos16, post16):
    grid = NSPAD // QBLK
    cols, ev = pl.pallas_call(
        _knn_body,
        grid=(grid,),
        in_specs=[
            pl.BlockSpec((QBLK, 128), lambda b: (b, 0)),
            pl.BlockSpec((16, NPAD), lambda b: (0, 0)),
        ],
        out_specs=[
            pl.BlockSpec((QBLK, 16), lambda b: (b, 0)),
            pl.BlockSpec((QBLK, 16), lambda b: (b, 0)),
        ],
        out_shape=[
            jax.ShapeDtypeStruct((NSPAD, 16), jnp.int32),
            jax.ShapeDtypeStruct((NSPAD, 16), jnp.int32),
        ],
        scratch_shapes=[pltpu.VMEM((QBLK, NPAD), jnp.float32)],
    )(qpos16, post16)
    return cols, ev


def _sc_gather(table, idx):
    B = idx.shape[0]
    bpw = B // 32
    nchunks = max(1, bpw // 400)
    csz = bpw // nchunks
    mesh = plsc.VectorSubcoreMesh(core_axis_name="c", subcore_axis_name="s")

    @functools.partial(
        pl.kernel, mesh=mesh,
        out_type=jax.ShapeDtypeStruct((B, 128), jnp.float32),
        scratch_types=[
            pltpu.VMEM((bpw,), jnp.int32),
            pltpu.VMEM((csz, 128), jnp.float32),
            pltpu.SemaphoreType.DMA,
        ],
    )
    def k(table_hbm, idx_hbm, out_hbm, idx_v, rows_v, sem):
        wid = lax.axis_index("s") * 2 + lax.axis_index("c")
        base = wid * bpw
        pltpu.sync_copy(idx_hbm.at[pl.ds(base, bpw)], idx_v)
        for c in range(nchunks):
            pltpu.async_copy(table_hbm.at[idx_v.at[pl.ds(c * csz, csz)]],
                             rows_v, sem).wait()
            pltpu.sync_copy(rows_v, out_hbm.at[pl.ds(base + c * csz, csz)])

    return k(table, idx)


def _conv_body(qp_ref, pc_ref, ev_ref,
               pw0_ref, pb0_ref, pw1_ref, pb1_ref, aw0_ref, ab0_ref,
               lw_ref, lsw_ref, ldw_ref, out_ref):
    qp = qp_ref[...]
    pw0 = pw0_ref[...]
    pb0 = pb0_ref[...]
    pw1 = pw1_ref[...]
    pb1 = pb1_ref[...]
    aw0 = aw0_ref[...]
    ab0 = ab0_ref[...]
    lw = lw_ref[...]
    lsw = lsw_ref[...]
    ldw = ldw_ref[...]

    adst = jnp.dot(qp, ldw, preferred_element_type=jnp.float32)

    alphas = []
    deltas = []
    xvs = []
    amax = None
    for j in range(KNB):
        pc = pc_ref[j][:, :16]
        asrc = jnp.dot(pc, lsw, preferred_element_type=jnp.float32)
        rel = qp - pc
        h = jnp.maximum(
            jnp.dot(rel, pw0, preferred_element_type=jnp.float32) + pb0, 0.0)
        delta = jnp.dot(h, pw1, preferred_element_type=jnp.float32) + pb1
        a = jnp.dot(adst - asrc + delta, aw0,
                    preferred_element_type=jnp.float32) + ab0
        evj = ev_ref[:, j:j + 1] > 0
        a = jnp.where(evj, a, -1e30)
        xv = jnp.dot(pc, lw, preferred_element_type=jnp.float32)
        alphas.append(a)
        deltas.append(delta)
        xvs.append(xv)
        amax = a if amax is None else jnp.maximum(amax, a)

    amax = jnp.where(amax > -1e29, amax, 0.0)
    den = jnp.zeros_like(amax)
    acc = jnp.zeros_like(amax)
    for j in range(KNB):
        ex = jnp.exp(alphas[j] - amax)
        den = den + ex
        acc = acc + ex * (xvs[j] + deltas[j])
    out_ref[...] = acc / jnp.maximum(den, 1e-16)


def _conv(qpos16, posc, ev, pw0p, pb0, pw1, pb1, aw0, ab0, lwp, lswp, ldwp):
    grid = NSPAD // CBLK
    out = pl.pallas_call(
        _conv_body,
        grid=(grid,),
        in_specs=[
            pl.BlockSpec((CBLK, 16), lambda b: (b, 0)),
            pl.BlockSpec((KNB, CBLK, 128), lambda b: (0, b, 0)),
            pl.BlockSpec((CBLK, 16), lambda b: (b, 0)),
            pl.BlockSpec((16, HID), lambda b: (0, 0)),
            pl.BlockSpec((1, HID), lambda b: (0, 0)),
            pl.BlockSpec((HID, OCH), lambda b: (0, 0)),
            pl.BlockSpec((1, OCH), lambda b: (0, 0)),
            pl.BlockSpec((OCH, OCH), lambda b: (0, 0)),
            pl.BlockSpec((1, OCH), lambda b: (0, 0)),
            pl.BlockSpec((16, OCH), lambda b: (0, 0)),
            pl.BlockSpec((16, OCH), lambda b: (0, 0)),
            pl.BlockSpec((16, OCH), lambda b: (0, 0)),
        ],
        out_specs=pl.BlockSpec((CBLK, OCH), lambda b: (b, 0)),
        out_shape=jax.ShapeDtypeStruct((NSPAD, OCH), jnp.float32),
    )(qpos16, posc, ev, pw0p, pb0, pw1, pb1, aw0, ab0, lwp, lswp, ldwp)
    return out


def kernel(pos, batch, pw0, pb0, pw1, pb1, aw0, ab0, lin_w, lin_src_w,
           lin_dst_w):
    pos = pos.astype(jnp.float32)

    idx = _fps(pos)

    pos128 = jnp.zeros((NPAD, 128), jnp.float32).at[:NPTS, :3].set(pos)
    post16 = jnp.zeros((16, NPAD), jnp.float32).at[:3, :NPTS].set(pos.T)
    idx_pad = jnp.zeros((NSPAD,), jnp.int32).at[:NSAMP].set(idx)
    qpos16 = _sc_gather(pos128, idx_pad)
    cols16, ev16 = _knn(qpos16, post16)

    cols_jq = cols16[:, :KNB].T.reshape(-1)
    posc = _sc_gather(pos128, cols_jq).reshape(KNB, NSPAD, 128)

    qp_conv = jnp.zeros((NSPAD, 16), jnp.float32).at[:, :3].set(pos[:NSPAD])
    pw0p = jnp.zeros((16, HID), jnp.float32).at[:3].set(pw0)
    lwp = jnp.zeros((16, OCH), jnp.float32).at[:3].set(lin_w)
    lswp = jnp.zeros((16, OCH), jnp.float32).at[:3].set(lin_src_w)
    ldwp = jnp.zeros((16, OCH), jnp.float32).at[:3].set(lin_dst_w)
    res = _conv(qp_conv, posc, ev16, pw0p, pb0[None, :], pw1, pb1[None, :],
                aw0, ab0[None, :], lwp, lswp, ldwp)

    out = jnp.zeros((NPTS, OCH), jnp.float32).at[:NSAMP].set(res[:NSAMP])
    return out


# speedup vs baseline: 16.9600x; 1.0674x over previous
import functools

import jax
import jax.numpy as jnp
from jax import lax
from jax.experimental import pallas as pl
from jax.experimental.pallas import tpu as pltpu
from jax.experimental.pallas import tpu_sc as plsc

NPTS = 10000
NPAD = 10240
NSAMP = 5000
NSPAD = 5120
RAD2 = 0.25 * 0.25
KNB = 10
HID = 64
OCH = 128
QBLK = 256
CBLK = 512


def _fps_body(p3_ref, ps_ref, idx_ref, d_ref):
    px = p3_ref[0]
    py = p3_ref[1]
    pz = p3_ref[2]
    rows = lax.broadcasted_iota(jnp.int32, (8, 1280), 0)
    cols = lax.broadcasted_iota(jnp.int32, (8, 1280), 1)
    fidx = rows * 1280 + cols
    inb = fidx < NPTS

    x0 = ps_ref[0, 0]
    y0 = ps_ref[1, 0]
    z0 = ps_ref[2, 0]
    dx = px - x0
    dy = py - y0
    dz = pz - z0
    d0 = (dx * dx + dy * dy) + dz * dz
    d_ref[...] = jnp.where(inb, d0, -jnp.inf)
    idx_ref[0] = jnp.int32(0)

    def body(i, carry):
        d = d_ref[...]
        m = jnp.max(d, axis=0, keepdims=True)
        m = jnp.max(m, axis=1, keepdims=True)
        eq = d == m
        key = jnp.where(eq, fidx, jnp.int32(NPAD))
        nk = jnp.min(key, axis=0, keepdims=True)
        nxt = jnp.min(nk, axis=1, keepdims=True)[0, 0]
        xs = ps_ref[0, nxt]
        ys = ps_ref[1, nxt]
        zs = ps_ref[2, nxt]
        ddx = px - xs
        ddy = py - ys
        ddz = pz - zs
        nd = (ddx * ddx + ddy * ddy) + ddz * ddz
        d_ref[...] = jnp.minimum(d, nd)
        idx_ref[i] = nxt
        return carry

    lax.fori_loop(1, NSAMP, body, 0, unroll=4)


def _fps(pos):
    p = jnp.zeros((3, NPAD), jnp.float32).at[:, :NPTS].set(pos.T)
    p3 = p.reshape(3, 8, 1280)
    idx = pl.pallas_call(
        _fps_body,
        out_shape=jax.ShapeDtypeStruct((NSAMP,), jnp.int32),
        in_specs=[pl.BlockSpec(memory_space=pltpu.VMEM),
                  pl.BlockSpec(memory_space=pltpu.SMEM)],
        out_specs=pl.BlockSpec(memory_space=pltpu.SMEM),
        scratch_shapes=[pltpu.VMEM((8, 1280), jnp.float32)],
    )(p3, p)
    return idx


def _knn_body(qp_ref, pt_ref, cols_ref, ev_ref, d2_ref):
    qp = qp_ref[...][:, :16]
    pt = pt_ref[...]
    q2 = jnp.sum(qp * qp, axis=1, keepdims=True)
    p2 = jnp.sum(pt * pt, axis=0, keepdims=True)
    dot = jnp.dot(qp, pt, preferred_element_type=jnp.float32)
    d2 = q2 + p2 - 2.0 * dot
    cidx = lax.broadcasted_iota(jnp.int32, (QBLK, NPAD), 1)
    valid = (d2 <= RAD2) & (cidx < NPTS)
    d2_ref[...] = jnp.where(valid, d2, jnp.inf)

    for k in range(KNB):
        d2m = d2_ref[...]
        mn = jnp.min(d2m, axis=1, keepdims=True)
        eq = d2m == mn
        c = jnp.min(jnp.where(eq, cidx, jnp.int32(NPAD)), axis=1,
                    keepdims=True)
        cols_ref[:, k:k + 1] = c
        ev_ref[:, k:k + 1] = (mn < jnp.inf).astype(jnp.int32)
        d2_ref[...] = jnp.where(cidx == c, jnp.inf, d2m)


def _knn(qpos16, post16):
    grid = NSPAD // QBLK
    cols, ev = pl.pallas_call(
        _knn_body,
        grid=(grid,),
        in_specs=[
            pl.BlockSpec((QBLK, 128), lambda b: (b, 0)),
            pl.BlockSpec((16, NPAD), lambda b: (0, 0)),
        ],
        out_specs=[
            pl.BlockSpec((QBLK, 16), lambda b: (b, 0)),
            pl.BlockSpec((QBLK, 16), lambda b: (b, 0)),
        ],
        out_shape=[
            jax.ShapeDtypeStruct((NSPAD, 16), jnp.int32),
            jax.ShapeDtypeStruct((NSPAD, 16), jnp.int32),
        ],
        scratch_shapes=[pltpu.VMEM((QBLK, NPAD), jnp.float32)],
    )(qpos16, post16)
    return cols, ev


def _sc_gather(table, idx):
    B = idx.shape[0]
    bpw = B // 32
    nchunks = max(1, bpw // 400)
    csz = bpw // nchunks
    mesh = plsc.VectorSubcoreMesh(core_axis_name="c", subcore_axis_name="s")

    @functools.partial(
        pl.kernel, mesh=mesh,
        out_type=jax.ShapeDtypeStruct((B, 128), jnp.float32),
        scratch_types=[
            pltpu.VMEM((bpw,), jnp.int32),
            pltpu.VMEM((csz, 128), jnp.float32),
            pltpu.SemaphoreType.DMA,
        ],
    )
    def k(table_hbm, idx_hbm, out_hbm, idx_v, rows_v, sem):
        wid = lax.axis_index("s") * 2 + lax.axis_index("c")
        base = wid * bpw
        pltpu.sync_copy(idx_hbm.at[pl.ds(base, bpw)], idx_v)
        for c in range(nchunks):
            pltpu.async_copy(table_hbm.at[idx_v.at[pl.ds(c * csz, csz)]],
                             rows_v, sem).wait()
            pltpu.sync_copy(rows_v, out_hbm.at[pl.ds(base + c * csz, csz)])

    return k(table, idx)


def _conv_body(qp_ref, pc_ref, ev_ref,
               pw0_ref, pb0_ref, pw1_ref, pb1_ref, aw0_ref, ab0_ref,
               lw_ref, lsw_ref, ldw_ref, out_ref):
    qp = qp_ref[...]
    pw0 = pw0_ref[...]
    pb0 = pb0_ref[...]
    pw1 = pw1_ref[...]
    pb1 = pb1_ref[...]
    aw0 = aw0_ref[...]
    ab0 = ab0_ref[...]
    lw = lw_ref[...]
    lsw = lsw_ref[...]
    ldw = ldw_ref[...]

    adst = jnp.dot(qp, ldw, preferred_element_type=jnp.float32)

    alphas = []
    deltas = []
    xvs = []
    amax = None
    for j in range(KNB):
        pc = pc_ref[j][:, :16]
        asrc = jnp.dot(pc, lsw, preferred_element_type=jnp.float32)
        rel = qp - pc
        h = jnp.maximum(
            jnp.dot(rel, pw0, preferred_element_type=jnp.float32) + pb0, 0.0)
        delta = jnp.dot(h, pw1, preferred_element_type=jnp.float32) + pb1
        a = jnp.dot(adst - asrc + delta, aw0,
                    preferred_element_type=jnp.float32) + ab0
        evj = ev_ref[:, j:j + 1] > 0
        a = jnp.where(evj, a, -1e30)
        xv = jnp.dot(pc, lw, preferred_element_type=jnp.float32)
        alphas.append(a)
        deltas.append(delta)
        xvs.append(xv)
        amax = a if amax is None else jnp.maximum(amax, a)

    amax = jnp.where(amax > -1e29, amax, 0.0)
    den = jnp.zeros_like(amax)
    acc = jnp.zeros_like(amax)
    for j in range(KNB):
        ex = jnp.exp(alphas[j] - amax)
        den = den + ex
        acc = acc + ex * (xvs[j] + deltas[j])
    out_ref[...] = acc / jnp.maximum(den, 1e-16)


def _conv(qpos16, posc, ev, pw0p, pb0, pw1, pb1, aw0, ab0, lwp, lswp, ldwp):
    grid = NSPAD // CBLK
    out = pl.pallas_call(
        _conv_body,
        grid=(grid,),
        in_specs=[
            pl.BlockSpec((CBLK, 16), lambda b: (b, 0)),
            pl.BlockSpec((KNB, CBLK, 128), lambda b: (0, b, 0)),
            pl.BlockSpec((CBLK, 16), lambda b: (b, 0)),
            pl.BlockSpec((16, HID), lambda b: (0, 0)),
            pl.BlockSpec((1, HID), lambda b: (0, 0)),
            pl.BlockSpec((HID, OCH), lambda b: (0, 0)),
            pl.BlockSpec((1, OCH), lambda b: (0, 0)),
            pl.BlockSpec((OCH, OCH), lambda b: (0, 0)),
            pl.BlockSpec((1, OCH), lambda b: (0, 0)),
            pl.BlockSpec((16, OCH), lambda b: (0, 0)),
            pl.BlockSpec((16, OCH), lambda b: (0, 0)),
            pl.BlockSpec((16, OCH), lambda b: (0, 0)),
        ],
        out_specs=pl.BlockSpec((CBLK, OCH), lambda b: (b, 0)),
        out_shape=jax.ShapeDtypeStruct((NSPAD, OCH), jnp.float32),
    )(qpos16, posc, ev, pw0p, pb0, pw1, pb1, aw0, ab0, lwp, lswp, ldwp)
    return out


def kernel(pos, batch, pw0, pb0, pw1, pb1, aw0, ab0, lin_w, lin_src_w,
           lin_dst_w):
    pos = pos.astype(jnp.float32)

    idx = _fps(pos)

    pos128 = jnp.zeros((NPAD, 128), jnp.float32).at[:NPTS, :3].set(pos)
    post16 = jnp.zeros((16, NPAD), jnp.float32).at[:3, :NPTS].set(pos.T)
    idx_pad = jnp.zeros((NSPAD,), jnp.int32).at[:NSAMP].set(idx)
    qpos16 = _sc_gather(pos128, idx_pad)
    cols16, ev16 = _knn(qpos16, post16)

    cols_jq = cols16[:, :KNB].T.reshape(-1)
    posc = _sc_gather(pos128, cols_jq).reshape(KNB, NSPAD, 128)

    qp_conv = jnp.zeros((NSPAD, 16), jnp.float32).at[:, :3].set(pos[:NSPAD])
    pw0p = jnp.zeros((16, HID), jnp.float32).at[:3].set(pw0)
    lwp = jnp.zeros((16, OCH), jnp.float32).at[:3].set(lin_w)
    lswp = jnp.zeros((16, OCH), jnp.float32).at[:3].set(lin_src_w)
    ldwp = jnp.zeros((16, OCH), jnp.float32).at[:3].set(lin_dst_w)
    res = _conv(qp_conv, posc, ev16, pw0p, pb0[None, :], pw1, pb1[None, :],
                aw0, ab0[None, :], lwp, lswp, ldwp)

    out = jnp.zeros((NPTS, OCH), jnp.float32).at[:NSAMP].set(res[:NSAMP])
    return out


# speedup vs baseline: 16.9855x; 1.0015x over previous
import functools

import jax
import jax.numpy as jnp
from jax import lax
from jax.experimental import pallas as pl
from jax.experimental.pallas import tpu as pltpu
from jax.experimental.pallas import tpu_sc as plsc

NPTS = 10000
NPAD = 10240
NSAMP = 5000
NSPAD = 5120
RAD2 = 0.25 * 0.25
KNB = 10
HID = 64
OCH = 128
QBLK = 256
CBLK = 512


def _fps_body(p3_ref, ps_ref, idx_ref, d_ref):
    px = p3_ref[0]
    py = p3_ref[1]
    pz = p3_ref[2]
    rows = lax.broadcasted_iota(jnp.int32, (8, 1280), 0)
    cols = lax.broadcasted_iota(jnp.int32, (8, 1280), 1)
    fidx = rows * 1280 + cols
    inb = fidx < NPTS

    x0 = ps_ref[0, 0]
    y0 = ps_ref[1, 0]
    z0 = ps_ref[2, 0]
    dx = px - x0
    dy = py - y0
    dz = pz - z0
    d0 = (dx * dx + dy * dy) + dz * dz
    idx_ref[0] = jnp.int32(0)

    def body(i, d):
        m = jnp.max(d, axis=0, keepdims=True)
        m = jnp.max(m, axis=1, keepdims=True)
        eq = d == m
        key = jnp.where(eq, fidx, jnp.int32(NPAD))
        nk = jnp.min(key, axis=0, keepdims=True)
        nxt = jnp.min(nk, axis=1, keepdims=True)[0, 0]
        xs = ps_ref[0, nxt]
        ys = ps_ref[1, nxt]
        zs = ps_ref[2, nxt]
        ddx = px - xs
        ddy = py - ys
        ddz = pz - zs
        nd = (ddx * ddx + ddy * ddy) + ddz * ddz
        idx_ref[i] = nxt
        return jnp.minimum(d, nd)

    d_ref[...] = lax.fori_loop(1, NSAMP, body,
                               jnp.where(inb, d0, -jnp.inf), unroll=4)


def _fps(pos):
    p = jnp.zeros((3, NPAD), jnp.float32).at[:, :NPTS].set(pos.T)
    p3 = p.reshape(3, 8, 1280)
    idx = pl.pallas_call(
        _fps_body,
        out_shape=jax.ShapeDtypeStruct((NSAMP,), jnp.int32),
        in_specs=[pl.BlockSpec(memory_space=pltpu.VMEM),
                  pl.BlockSpec(memory_space=pltpu.SMEM)],
        out_specs=pl.BlockSpec(memory_space=pltpu.SMEM),
        scratch_shapes=[pltpu.VMEM((8, 1280), jnp.float32)],
    )(p3, p)
    return idx


def _knn_body(qp_ref, pt_ref, cols_ref, ev_ref, d2_ref):
    qp = qp_ref[...][:, :16]
    pt = pt_ref[...]
    q2 = jnp.sum(qp * qp, axis=1, keepdims=True)
    p2 = jnp.sum(pt * pt, axis=0, keepdims=True)
    dot = jnp.dot(qp, pt, preferred_element_type=jnp.float32)
    d2 = q2 + p2 - 2.0 * dot
    cidx = lax.broadcasted_iota(jnp.int32, (QBLK, NPAD), 1)
    valid = (d2 <= RAD2) & (cidx < NPTS)
    d2_ref[...] = jnp.where(valid, d2, jnp.inf)

    for k in range(KNB):
        d2m = d2_ref[...]
        mn = jnp.min(d2m, axis=1, keepdims=True)
        eq = d2m == mn
        c = jnp.min(jnp.where(eq, cidx, jnp.int32(NPAD)), axis=1,
                    keepdims=True)
        cols_ref[:, k:k + 1] = c
        ev_ref[:, k:k + 1] = (mn < jnp.inf).astype(jnp.int32)
        d2_ref[...] = jnp.where(cidx == c, jnp.inf, d2m)


def _knn(qpos16, post16):
    grid = NSPAD // QBLK
    cols, ev = pl.pallas_call(
        _knn_body,
        grid=(grid,),
        in_specs=[
            pl.BlockSpec((QBLK, 128), lambda b: (b, 0)),
            pl.BlockSpec((16, NPAD), lambda b: (0, 0)),
        ],
        out_specs=[
            pl.BlockSpec((QBLK, 16), lambda b: (b, 0)),
            pl.BlockSpec((QBLK, 16), lambda b: (b, 0)),
        ],
        out_shape=[
            jax.ShapeDtypeStruct((NSPAD, 16), jnp.int32),
            jax.ShapeDtypeStruct((NSPAD, 16), jnp.int32),
        ],
        scratch_shapes=[pltpu.VMEM((QBLK, NPAD), jnp.float32)],
    )(qpos16, post16)
    return cols, ev


def _sc_gather(table, idx):
    B = idx.shape[0]
    bpw = B // 32
    nchunks = max(1, bpw // 400)
    csz = bpw // nchunks
    mesh = plsc.VectorSubcoreMesh(core_axis_name="c", subcore_axis_name="s")

    @functools.partial(
        pl.kernel, mesh=mesh,
        out_type=jax.ShapeDtypeStruct((B, 128), jnp.float32),
        scratch_types=[
            pltpu.VMEM((bpw,), jnp.int32),
            pltpu.VMEM((csz, 128), jnp.float32),
            pltpu.SemaphoreType.DMA,
        ],
    )
    def k(table_hbm, idx_hbm, out_hbm, idx_v, rows_v, sem):
        wid = lax.axis_index("s") * 2 + lax.axis_index("c")
        base = wid * bpw
        pltpu.sync_copy(idx_hbm.at[pl.ds(base, bpw)], idx_v)
        for c in range(nchunks):
            pltpu.async_copy(table_hbm.at[idx_v.at[pl.ds(c * csz, csz)]],
                             rows_v, sem).wait()
            pltpu.sync_copy(rows_v, out_hbm.at[pl.ds(base + c * csz, csz)])

    return k(table, idx)


def _conv_body(qp_ref, pc_ref, ev_ref,
               pw0_ref, pb0_ref, pw1_ref, pb1_ref, aw0_ref, ab0_ref,
               lw_ref, lsw_ref, ldw_ref, out_ref):
    qp = qp_ref[...]
    pw0 = pw0_ref[...]
    pb0 = pb0_ref[...]
    pw1 = pw1_ref[...]
    pb1 = pb1_ref[...]
    aw0 = aw0_ref[...]
    ab0 = ab0_ref[...]
    lw = lw_ref[...]
    lsw = lsw_ref[...]
    ldw = ldw_ref[...]

    adst = jnp.dot(qp, ldw, preferred_element_type=jnp.float32)

    alphas = []
    deltas = []
    xvs = []
    amax = None
    for j in range(KNB):
        pc = pc_ref[j][:, :16]
        asrc = jnp.dot(pc, lsw, preferred_element_type=jnp.float32)
        rel = qp - pc
        h = jnp.maximum(
            jnp.dot(rel, pw0, preferred_element_type=jnp.float32) + pb0, 0.0)
        delta = jnp.dot(h, pw1, preferred_element_type=jnp.float32) + pb1
        a = jnp.dot(adst - asrc + delta, aw0,
                    preferred_element_type=jnp.float32) + ab0
        evj = ev_ref[:, j:j + 1] > 0
        a = jnp.where(evj, a, -1e30)
        xv = jnp.dot(pc, lw, preferred_element_type=jnp.float32)
        alphas.append(a)
        deltas.append(delta)
        xvs.append(xv)
        amax = a if amax is None else jnp.maximum(amax, a)

    amax = jnp.where(amax > -1e29, amax, 0.0)
    den = jnp.zeros_like(amax)
    acc = jnp.zeros_like(amax)
    for j in range(KNB):
        ex = jnp.exp(alphas[j] - amax)
        den = den + ex
        acc = acc + ex * (xvs[j] + deltas[j])
    out_ref[...] = acc / jnp.maximum(den, 1e-16)


def _conv(qpos16, posc, ev, pw0p, pb0, pw1, pb1, aw0, ab0, lwp, lswp, ldwp):
    grid = NSPAD // CBLK
    out = pl.pallas_call(
        _conv_body,
        grid=(grid,),
        in_specs=[
            pl.BlockSpec((CBLK, 16), lambda b: (b, 0)),
            pl.BlockSpec((KNB, CBLK, 128), lambda b: (0, b, 0)),
            pl.BlockSpec((CBLK, 16), lambda b: (b, 0)),
            pl.BlockSpec((16, HID), lambda b: (0, 0)),
            pl.BlockSpec((1, HID), lambda b: (0, 0)),
            pl.BlockSpec((HID, OCH), lambda b: (0, 0)),
            pl.BlockSpec((1, OCH), lambda b: (0, 0)),
            pl.BlockSpec((OCH, OCH), lambda b: (0, 0)),
            pl.BlockSpec((1, OCH), lambda b: (0, 0)),
            pl.BlockSpec((16, OCH), lambda b: (0, 0)),
            pl.BlockSpec((16, OCH), lambda b: (0, 0)),
            pl.BlockSpec((16, OCH), lambda b: (0, 0)),
        ],
        out_specs=pl.BlockSpec((CBLK, OCH), lambda b: (b, 0)),
        out_shape=jax.ShapeDtypeStruct((NSPAD, OCH), jnp.float32),
    )(qpos16, posc, ev, pw0p, pb0, pw1, pb1, aw0, ab0, lwp, lswp, ldwp)
    return out


def kernel(pos, batch, pw0, pb0, pw1, pb1, aw0, ab0, lin_w, lin_src_w,
           lin_dst_w):
    pos = pos.astype(jnp.float32)

    idx = _fps(pos)

    pos128 = jnp.zeros((NPAD, 128), jnp.float32).at[:NPTS, :3].set(pos)
    post16 = jnp.zeros((16, NPAD), jnp.float32).at[:3, :NPTS].set(pos.T)
    idx_pad = jnp.zeros((NSPAD,), jnp.int32).at[:NSAMP].set(idx)
    qpos16 = _sc_gather(pos128, idx_pad)
    cols16, ev16 = _knn(qpos16, post16)

    cols_jq = cols16[:, :KNB].T.reshape(-1)
    posc = _sc_gather(pos128, cols_jq).reshape(KNB, NSPAD, 128)

    qp_conv = jnp.zeros((NSPAD, 16), jnp.float32).at[:, :3].set(pos[:NSPAD])
    pw0p = jnp.zeros((16, HID), jnp.float32).at[:3].set(pw0)
    lwp = jnp.zeros((16, OCH), jnp.float32).at[:3].set(lin_w)
    lswp = jnp.zeros((16, OCH), jnp.float32).at[:3].set(lin_src_w)
    ldwp = jnp.zeros((16, OCH), jnp.float32).at[:3].set(lin_dst_w)
    res = _conv(qp_conv, posc, ev16, pw0p, pb0[None, :], pw1, pb1[None, :],
                aw0, ab0[None, :], lwp, lswp, ldwp)

    out = jnp.zeros((NPTS, OCH), jnp.float32).at[:NSAMP].set(res[:NSAMP])
    return out
